# baseline (speedup 1.0000x reference)
import jax
import jax.numpy as jnp
import numpy as np
from jax.experimental import pallas as pl
from jax.experimental.pallas import tpu as pltpu

N = 512
E = 16384
F_IN = 128
H = 256
LS = 32
LY = 32
NL = 8
NUM_EDGES = N * (N - 1) // 2 + N

P1 = 8
CB2 = 6912
P2 = NUM_EDGES // CB2
GRID = 1 + P1 + P2
WIN = N + 128

_INTERPRET = False


def _tri_schedule():
    tri = lambda i: i * (i + 1) // 2
    avail, i = [], 0
    for b in range(P2):
        hi = CB2 * (b + 1)
        while i < N and tri(i) + i + 1 <= hi:
            i += 1
        avail.append(i)
    cum, prev = [], 0
    for b in range(P2):
        prev = min(avail[b], max(prev, 27 * (b + 1)))
        cum.append(prev)
    cum[-1] = N
    return np.array([0] * (2 + P1) + cum, dtype=np.int32)


def _body(sched_ref, ei_ref, x_ref, y_ref, eps_s_ref, eps_y_ref,
          us1w, us1b, usmuw, usmub, uslogw, uslogb,
          uy1w, uy1b, uymuw, uymub, uylogw, uylogb,
          sd1w, sd1b, sd2w, sd2b,
          xd1w, xd1b, xd2w, xd2b,
          yd1w, yd1b, yd2w, yd2b,
          yp1w, yp1b, yp2w, yp2b,
          w3_ref, b1_ref, w2_ref, b2_ref,
          mu_s_o, log_s_o, mu_y_o, log_y_o,
          s_hat_o, xp_o, ypred_o, yprime_o, l_ref, a_ref,
          latT_ref, l1_ref):
    f32 = jnp.float32
    j = pl.program_id(0)

    @pl.when(j == 0)
    def _fwd():
        iota_n = jax.lax.broadcasted_iota(jnp.int32, (N, 1), 0)
        C = jnp.zeros((N, N), f32)
        CHUNK = 2048
        for k in range(E // CHUNK):
            src = ei_ref[0:1, k * CHUNK:(k + 1) * CHUNK]
            dst = ei_ref[1:2, k * CHUNK:(k + 1) * CHUNK]
            osrc_t = (iota_n == src).astype(jnp.bfloat16)
            odst_t = (iota_n == dst).astype(jnp.bfloat16)
            C = C + jax.lax.dot_general(
                odst_t, osrc_t, (((1,), (1,)), ((), ())),
                preferred_element_type=f32)
        eye = (iota_n == jax.lax.broadcasted_iota(jnp.int32, (1, N), 1)
               ).astype(f32)
        C = C + eye
        deg = jnp.sum(C, axis=1, keepdims=True)
        dinv = jax.lax.rsqrt(deg)
        A = C * dinv * dinv.reshape(1, N)

        x = x_ref[...]

        def mm(a, b):
            return jnp.dot(a, b, preferred_element_type=f32)

        def g(h, w_ref, b_ref):
            return mm(A, mm(h, w_ref[...])) + b_ref[...]

        h = jax.nn.relu(g(x, us1w, us1b))
        mu_s = g(h, usmuw, usmub)
        log_s = g(h, uslogw, uslogb)
        mu_s_o[...] = mu_s
        log_s_o[...] = log_s
        xw2 = mm(x, uy1w[0:F_IN, :]) + y_ref[...] * uy1w[F_IN:F_IN + 1, :]
        h2 = jax.nn.relu(mm(A, xw2) + uy1b[...])
        mu_y = g(h2, uymuw, uymub)
        log_y = g(h2, uylogw, uylogb)
        mu_y_o[...] = mu_y
        log_y_o[...] = log_y
        u_s = eps_s_ref[...] * jnp.exp(0.5 * log_s) + mu_s
        u_y = eps_y_ref[...] * jnp.exp(0.5 * log_y) + mu_y
        latT_ref[0:LS, :] = u_s.T
        latT_ref[LS:LS + LY, :] = u_y.T
        l1_ref[...] = b1_ref[...]
        s1 = jax.nn.relu(g(u_s, sd1w, sd1b))
        s2 = jax.nn.relu(g(s1, sd2w, sd2b))
        s_hat_o[...] = jax.nn.sigmoid(s2)
        xw3 = mm(u_s, xd1w[0:LS, :]) + mm(u_y, xd1w[LS:LS + LY, :])
        xp1 = mm(A, xw3) + xd1b[...]
        xp = g(xp1, xd2w, xd2b)
        xp_o[...] = xp
        xw4 = mm(xp, yd1w[0:F_IN, :]) + mm(u_y, yd1w[F_IN:F_IN + LY, :])
        yl1 = mm(A, xw4) + yd1b[...]
        ypred_o[...] = jax.nn.softmax(g(yl1, yd2w, yd2b), axis=1)
        yq1 = g(x, yp1w, yp1b)
        yprime_o[...] = jax.nn.softmax(g(yq1, yp2w, yp2b), axis=1)

    @pl.when((j >= 1) & (j <= P1))
    def _fc1():
        acc = jnp.zeros((1, 512), jnp.float32)
        for cc in range(8):
            c = 8 * (j - 1) + cc
            acc += jnp.dot(latT_ref[pl.ds(c, 1), :], w3_ref[:, cc, :],
                           preferred_element_type=jnp.float32)
        l1_ref[...] += acc

    @pl.when(j > P1)
    def _fc2():
        b = j - (P1 + 1)
        val = jnp.dot(l1_ref[...], w2_ref[...],
                      preferred_element_type=jnp.float32) + b2_ref[...]
        l_ref[0:1, pl.ds(pl.multiple_of(b * CB2, 128), CB2)] = val
        iota_l = jax.lax.broadcasted_iota(jnp.int32, (1, N), 1)

        def rowbody(i, _):
            start = i * (i + 1) // 2
            base = jnp.minimum((start // 128) * 128, NUM_EDGES - WIN)
            base = pl.multiple_of(base, 128)
            off = start - base
            w = l_ref[0:1, pl.ds(base, WIN)]
            rowv = pltpu.roll(w, (WIN - off) % WIN, axis=1)[:, :N]
            a_ref[pl.ds(i, 1), :] = jnp.where(iota_l <= i, rowv, 0.0)
            return 0

        jax.lax.fori_loop(sched_ref[j], sched_ref[j + 1], rowbody, 0)


def kernel(x, edge_index, Y,
           us1_W, us1_b, usmu_W, usmu_b, uslog_W, uslog_b,
           uy1_W, uy1_b, uymu_W, uymu_b, uylog_W, uylog_b,
           sd1_W, sd1_b, sd2_W, sd2_b,
           xd1_W, xd1_b, xd2_W, xd2_b,
           fc1A_W, fc1A_b, fc2A_W, fc2A_b,
           yd1_W, yd1_b, yd2_W, yd2_b,
           yp1_W, yp1_b, yp2_W, yp2_b):
    f32 = jnp.float32
    row = lambda b: b.reshape(1, -1)
    eps_s = jax.random.normal(jax.random.key(42), (N, LS), dtype=f32)
    eps_y = jax.random.normal(jax.random.key(43), (N, LY), dtype=f32)
    sched = jnp.asarray(_tri_schedule())
    W3 = fc1A_W.reshape(512, LS + LY, 512)

    def const(shape):
        zeros = tuple(0 for _ in shape)
        return pl.BlockSpec(shape, lambda j, z=zeros: z)

    params = (
        us1_W, row(us1_b), usmu_W, row(usmu_b), uslog_W, row(uslog_b),
        uy1_W, row(uy1_b), uymu_W, row(uymu_b), uylog_W, row(uylog_b),
        sd1_W, row(sd1_b), sd2_W, row(sd2_b),
        xd1_W, row(xd1_b), xd2_W, row(xd2_b),
        yd1_W, row(yd1_b), yd2_W, row(yd2_b),
        yp1_W, row(yp1_b), yp2_W, row(yp2_b),
    )
    in_specs = (
        [pl.BlockSpec(memory_space=pltpu.SMEM)]
        + [const((2, E)), const((N, F_IN)), const((N, 1)),
           const((N, LS)), const((N, LY))]
        + [const(p.shape) for p in params]
        + [
            pl.BlockSpec((512, 8, 512),
                         lambda j: (0, jnp.clip(j - 1, 0, P1 - 1), 0)),
            const((1, 512)),
            pl.BlockSpec((512, CB2),
                         lambda j: (0, jnp.clip(j - (P1 + 1), 0, P2 - 1))),
            pl.BlockSpec((1, CB2),
                         lambda j: (0, jnp.clip(j - (P1 + 1), 0, P2 - 1))),
        ]
    )
    out_specs = (
        const((N, LS)), const((N, LS)), const((N, LY)), const((N, LY)),
        const((N, 1)), const((N, F_IN)), const((N, NL)), const((N, NL)),
        const((1, NUM_EDGES)), const((N, N)),
    )
    out_shape = (
        jax.ShapeDtypeStruct((N, LS), f32),
        jax.ShapeDtypeStruct((N, LS), f32),
        jax.ShapeDtypeStruct((N, LY), f32),
        jax.ShapeDtypeStruct((N, LY), f32),
        jax.ShapeDtypeStruct((N, 1), f32),
        jax.ShapeDtypeStruct((N, F_IN), f32),
        jax.ShapeDtypeStruct((N, NL), f32),
        jax.ShapeDtypeStruct((N, NL), f32),
        jax.ShapeDtypeStruct((1, NUM_EDGES), f32),
        jax.ShapeDtypeStruct((N, N), f32),
    )
    (mu_s, log_s, mu_y, log_y, s_hat, xp, ypred, yprime, l2d, A) = \
        pl.pallas_call(
            _body,
            grid=(GRID,),
            in_specs=in_specs,
            out_specs=out_specs,
            out_shape=out_shape,
            scratch_shapes=[pltpu.VMEM((LS + LY, N), f32),
                            pltpu.VMEM((1, 512), f32)],
            interpret=_INTERPRET,
        )(sched, edge_index, x, Y, eps_s, eps_y, *params,
          W3, row(fc1A_b), fc2A_W, row(fc2A_b))
    return (xp, A, l2d.reshape(NUM_EDGES), ypred, yprime,
            s_hat, mu_s, log_s, mu_y, log_y)


# speedup vs baseline: 1.0228x; 1.0228x over previous
import jax
import jax.numpy as jnp
import numpy as np
from jax.experimental import pallas as pl
from jax.experimental.pallas import tpu as pltpu

N = 512
E = 16384
F_IN = 128
H = 256
LS = 32
LY = 32
NL = 8
NUM_EDGES = N * (N - 1) // 2 + N

KB1 = 4096
P1 = (N * (LS + LY)) // KB1
CB2 = 6912
P2 = NUM_EDGES // CB2
WIN = N + 128

_INTERPRET = False


def _tri_schedule():
    tri = lambda i: i * (i + 1) // 2
    avail, i = [], 0
    for b in range(P2):
        hi = CB2 * (b + 1)
        while i < N and tri(i) + i + 1 <= hi:
            i += 1
        avail.append(i)
    cum, prev = [], 0
    for b in range(P2):
        prev = min(avail[b], max(prev, 27 * (b + 1)))
        cum.append(prev)
    cum[-1] = N
    return np.array([0] * (P1 + 1) + cum, dtype=np.int32)


_US1, _UY1A, _UY1B, _SD1, _XD1A, _XD1B = 0, 128, 256, 264, 296, 328
_YD1A, _YD1B, _YP1 = 0, 128, 160
_BOFF = {"us1": 0, "uy1": 256, "usmu": 512, "uslog": 640, "uymu": 768,
         "uylog": 896, "sd1": 1024, "sd2": 1280, "xd1": 1408, "xd2": 1664,
         "yd1": 1792, "yd2": 2304, "yp1": 2432, "yp2": 2944}


def _fwd_body(ei_ref, x_ref, y_ref, eps_s_ref, eps_y_ref,
              wp256, wp32, wp512, wpxs, wpyy, bp,
              mu_s_o, log_s_o, mu_y_o, log_y_o,
              s_hat_o, xp_o, ypred_o, yprime_o, lat_o):
    f32 = jnp.float32
    iota_n = jax.lax.broadcasted_iota(jnp.int32, (N, 1), 0)
    C = jnp.zeros((N, N), f32)
    CHUNK = 2048
    for k in range(E // CHUNK):
        src = ei_ref[0:1, k * CHUNK:(k + 1) * CHUNK]
        dst = ei_ref[1:2, k * CHUNK:(k + 1) * CHUNK]
        osrc_t = (iota_n == src).astype(jnp.bfloat16)
        odst_t = (iota_n == dst).astype(jnp.bfloat16)
        C = C + jax.lax.dot_general(
            odst_t, osrc_t, (((1,), (1,)), ((), ())),
            preferred_element_type=f32)
    eye = (iota_n == jax.lax.broadcasted_iota(jnp.int32, (1, N), 1)
           ).astype(f32)
    C = C + eye
    deg = jnp.sum(C, axis=1, keepdims=True)
    dinv = jax.lax.rsqrt(deg)
    A = C * dinv * dinv.reshape(1, N)

    x = x_ref[...]

    def mm(a, b):
        return jnp.dot(a, b, preferred_element_type=f32)

    def bias(name, n):
        o = _BOFF[name]
        return bp[0:1, o:o + n]

    h = jax.nn.relu(mm(A, mm(x, wp256[_US1:_US1 + F_IN, :]))
                    + bias("us1", H))
    mu_s = mm(A, mm(h, wp32[0:256, :])) + bias("usmu", LS)
    log_s = mm(A, mm(h, wp32[256:512, :])) + bias("uslog", LS)
    mu_s_o[...] = mu_s
    log_s_o[...] = log_s
    xw2 = mm(x, wp256[_UY1A:_UY1A + F_IN, :]) \
        + y_ref[...] * wp256[_UY1B:_UY1B + 1, :]
    h2 = jax.nn.relu(mm(A, xw2) + bias("uy1", H))
    mu_y = mm(A, mm(h2, wp32[512:768, :])) + bias("uymu", LY)
    log_y = mm(A, mm(h2, wp32[768:1024, :])) + bias("uylog", LY)
    mu_y_o[...] = mu_y
    log_y_o[...] = log_y
    u_s = eps_s_ref[...] * jnp.exp(0.5 * log_s) + mu_s
    u_y = eps_y_ref[...] * jnp.exp(0.5 * log_y) + mu_y
    lat_o[...] = jnp.concatenate([u_s, u_y], axis=1)
    s1 = jax.nn.relu(mm(A, mm(u_s, wp256[_SD1:_SD1 + LS, :]))
                     + bias("sd1", H))
    s2 = jax.nn.relu(mm(A, mm(s1, wpxs[:, 128:129])) + bias("sd2", 1))
    s_hat_o[...] = jax.nn.sigmoid(s2)
    xw3 = mm(u_s, wp256[_XD1A:_XD1A + LS, :]) \
        + mm(u_y, wp256[_XD1B:_XD1B + LY, :])
    xp1 = mm(A, xw3) + bias("xd1", H)
    xp = mm(A, mm(xp1, wpxs[:, 0:128])) + bias("xd2", F_IN)
    xp_o[...] = xp
    xw4 = mm(xp, wp512[_YD1A:_YD1A + F_IN, :]) \
        + mm(u_y, wp512[_YD1B:_YD1B + LY, :])
    yl1 = mm(A, xw4) + bias("yd1", 512)
    ylog = mm(A, mm(yl1, wpyy[:, 0:NL])) + bias("yd2", NL)
    ypred_o[...] = jax.nn.softmax(ylog, axis=1)
    yq1 = mm(A, mm(x, wp512[_YP1:_YP1 + F_IN, :])) + bias("yp1", 512)
    qlog = mm(A, mm(yq1, wpyy[:, NL:2 * NL])) + bias("yp2", NL)
    yprime_o[...] = jax.nn.softmax(qlog, axis=1)


def _run_fwd(edge_index, x, Y, eps_s, eps_y, params):
    f32 = jnp.float32
    outs = (
        jax.ShapeDtypeStruct((N, LS), f32),
        jax.ShapeDtypeStruct((N, LS), f32),
        jax.ShapeDtypeStruct((N, LY), f32),
        jax.ShapeDtypeStruct((N, LY), f32),
        jax.ShapeDtypeStruct((N, 1), f32),
        jax.ShapeDtypeStruct((N, F_IN), f32),
        jax.ShapeDtypeStruct((N, NL), f32),
        jax.ShapeDtypeStruct((N, NL), f32),
        jax.ShapeDtypeStruct((N, LS + LY), f32),
    )
    return pl.pallas_call(
        _fwd_body,
        out_shape=outs,
        interpret=_INTERPRET,
    )(edge_index, x, Y, eps_s, eps_y, *params)


def _fc_body(sched_ref, f_ref, w1_ref, b1_ref, w2_ref, b2_ref,
             l_ref, a_ref, l1_ref):
    j = pl.program_id(0)

    @pl.when(j == 0)
    def _():
        l1_ref[...] = b1_ref[...]

    @pl.when(j < P1)
    def _():
        l1_ref[...] += jnp.dot(f_ref[...], w1_ref[...],
                               preferred_element_type=jnp.float32)

    @pl.when(j >= P1)
    def _():
        b = j - P1
        val = jnp.dot(l1_ref[...], w2_ref[...],
                      preferred_element_type=jnp.float32) + b2_ref[...]
        l_ref[0:1, pl.ds(pl.multiple_of(b * CB2, 128), CB2)] = val
        iota_l = jax.lax.broadcasted_iota(jnp.int32, (1, N), 1)

        def rowbody(i, _):
            start = i * (i + 1) // 2
            base = jnp.minimum((start // 128) * 128, NUM_EDGES - WIN)
            base = pl.multiple_of(base, 128)
            off = start - base
            w = l_ref[0:1, pl.ds(base, WIN)]
            rowv = pltpu.roll(w, (WIN - off) % WIN, axis=1)[:, :N]
            a_ref[pl.ds(i, 1), :] = jnp.where(iota_l <= i, rowv, 0.0)
            return 0

        jax.lax.fori_loop(sched_ref[j], sched_ref[j + 1], rowbody, 0)


def _run_fc(sched, feat, W1, b1, W2, b2):
    f32 = jnp.float32
    return pl.pallas_call(
        _fc_body,
        grid=(P1 + P2,),
        in_specs=[
            pl.BlockSpec(memory_space=pltpu.SMEM),
            pl.BlockSpec((1, KB1), lambda j: (0, jnp.minimum(j, P1 - 1))),
            pl.BlockSpec((KB1, 512), lambda j: (jnp.minimum(j, P1 - 1), 0)),
            pl.BlockSpec((1, 512), lambda j: (0, 0)),
            pl.BlockSpec((512, CB2),
                         lambda j: (0, jnp.clip(j - P1, 0, P2 - 1))),
            pl.BlockSpec((1, CB2),
                         lambda j: (0, jnp.clip(j - P1, 0, P2 - 1))),
        ],
        out_specs=(
            pl.BlockSpec((1, NUM_EDGES), lambda j: (0, 0)),
            pl.BlockSpec((N, N), lambda j: (0, 0)),
        ),
        out_shape=(
            jax.ShapeDtypeStruct((1, NUM_EDGES), f32),
            jax.ShapeDtypeStruct((N, N), f32),
        ),
        scratch_shapes=[pltpu.VMEM((1, 512), f32)],
        interpret=_INTERPRET,
    )(sched, feat, W1, b1, W2, b2)


def kernel(x, edge_index, Y,
           us1_W, us1_b, usmu_W, usmu_b, uslog_W, uslog_b,
           uy1_W, uy1_b, uymu_W, uymu_b, uylog_W, uylog_b,
           sd1_W, sd1_b, sd2_W, sd2_b,
           xd1_W, xd1_b, xd2_W, xd2_b,
           fc1A_W, fc1A_b, fc2A_W, fc2A_b,
           yd1_W, yd1_b, yd2_W, yd2_b,
           yp1_W, yp1_b, yp2_W, yp2_b):
    f32 = jnp.float32
    row = lambda b: b.reshape(1, -1)
    eps_s = jax.random.normal(jax.random.key(42), (N, LS), dtype=f32)
    eps_y = jax.random.normal(jax.random.key(43), (N, LY), dtype=f32)

    z7 = jnp.zeros((7, H), f32)
    wp256 = jnp.concatenate(
        [us1_W, uy1_W[:F_IN], uy1_W[F_IN:F_IN + 1], z7,
         sd1_W, xd1_W[:LS], xd1_W[LS:]], axis=0)
    wp32 = jnp.concatenate([usmu_W, uslog_W, uymu_W, uylog_W],
                           axis=0)
    wp512 = jnp.concatenate([yd1_W[:F_IN], yd1_W[F_IN:], yp1_W],
                            axis=0)
    wpxs = jnp.concatenate([xd2_W, sd2_W], axis=1)
    wpyy = jnp.concatenate([yd2_W, yp2_W], axis=1)
    zf = lambda n: jnp.zeros((n,), f32)
    bp = jnp.concatenate(
        [us1_b, uy1_b, usmu_b, zf(96), uslog_b, zf(96), uymu_b, zf(96),
         uylog_b, zf(96), sd1_b, sd2_b, zf(127), xd1_b, xd2_b,
         yd1_b, yd2_b, zf(120), yp1_b, yp2_b, zf(120)]).reshape(1, 3072)
    params = (wp256, wp32, wp512, wpxs, wpyy, bp)
    (mu_s, log_s, mu_y, log_y, s_hat, xp, ypred, yprime, lat) = _run_fwd(
        edge_index, x, Y, eps_s, eps_y, params)

    feat = lat.reshape(1, N * (LS + LY))
    sched = jnp.asarray(_tri_schedule())
    l2d, A = _run_fc(sched, feat, fc1A_W, row(fc1A_b), fc2A_W, row(fc2A_b))
    return (xp, A, l2d.reshape(NUM_EDGES), ypred, yprime,
            s_hat, mu_s, log_s, mu_y, log_y)


# speedup vs baseline: 1.0230x; 1.0001x over previous
import jax
import jax.numpy as jnp
import numpy as np
from jax.experimental import pallas as pl
from jax.experimental.pallas import tpu as pltpu

N = 512
E = 16384
F_IN = 128
H = 256
LS = 32
LY = 32
NL = 8
NUM_EDGES = N * (N - 1) // 2 + N

KB1 = 4096
P1 = (N * (LS + LY)) // KB1
CB2 = 6912
P2 = NUM_EDGES // CB2
WIN = N + 128

_INTERPRET = False


def _tri_schedule():
    tri = lambda i: i * (i + 1) // 2
    avail, i = [], 0
    for b in range(P2):
        hi = CB2 * (b + 1)
        while i < N and tri(i) + i + 1 <= hi:
            i += 1
        avail.append(i)
    cum, prev = [], 0
    for b in range(P2):
        prev = min(avail[b], max(prev, 27 * (b + 1)))
        cum.append(prev)
    cum[-1] = N
    return np.array([0] * (P1 + 1) + cum, dtype=np.int32)


_US1, _UY1A, _UY1B, _SD1, _XD1A, _XD1B = 0, 128, 256, 264, 296, 328
_YD1A, _YD1B, _YP1 = 0, 128, 160
_BOFF = {"us1": 0, "uy1": 256, "usmu": 512, "uslog": 640, "uymu": 768,
         "uylog": 896, "sd1": 1024, "sd2": 1280, "xd1": 1408, "xd2": 1664,
         "yd1": 1792, "yd2": 2304, "yp1": 2432, "yp2": 2944}


def _fwd_body(ei_ref, x_ref, y_ref, eps_s_ref, eps_y_ref,
              wp256, wp32, wp512, wpxs, wpyy, bp,
              mu_s_o, log_s_o, mu_y_o, log_y_o,
              s_hat_o, xp_o, ypred_o, yprime_o, lat_o):
    f32 = jnp.float32
    iota_n = jax.lax.broadcasted_iota(jnp.int32, (N, 1), 0)
    C = jnp.zeros((N, N), f32)
    CHUNK = 2048
    for k in range(E // CHUNK):
        src = ei_ref[0:1, k * CHUNK:(k + 1) * CHUNK]
        dst = ei_ref[1:2, k * CHUNK:(k + 1) * CHUNK]
        osrc_t = (iota_n == src).astype(jnp.bfloat16)
        odst_t = (iota_n == dst).astype(jnp.bfloat16)
        C = C + jax.lax.dot_general(
            odst_t, osrc_t, (((1,), (1,)), ((), ())),
            preferred_element_type=f32)
    eye = (iota_n == jax.lax.broadcasted_iota(jnp.int32, (1, N), 1)
           ).astype(f32)
    C = C + eye
    deg = jnp.sum(C, axis=1, keepdims=True)
    dinv = jax.lax.rsqrt(deg)
    A = C * dinv * dinv.reshape(1, N)

    x = x_ref[...]

    def mm(a, b):
        return jnp.dot(a.astype(jnp.bfloat16), b.astype(jnp.bfloat16),
                       preferred_element_type=f32)

    def bias(name, n):
        o = _BOFF[name]
        return bp[0:1, o:o + n]

    h = jax.nn.relu(mm(A, mm(x, wp256[_US1:_US1 + F_IN, :]))
                    + bias("us1", H))
    mu_s = mm(A, mm(h, wp32[0:256, :])) + bias("usmu", LS)
    log_s = mm(A, mm(h, wp32[256:512, :])) + bias("uslog", LS)
    mu_s_o[...] = mu_s
    log_s_o[...] = log_s
    xw2 = mm(x, wp256[_UY1A:_UY1A + F_IN, :]) \
        + y_ref[...] * wp256[_UY1B:_UY1B + 1, :]
    h2 = jax.nn.relu(mm(A, xw2) + bias("uy1", H))
    mu_y = mm(A, mm(h2, wp32[512:768, :])) + bias("uymu", LY)
    log_y = mm(A, mm(h2, wp32[768:1024, :])) + bias("uylog", LY)
    mu_y_o[...] = mu_y
    log_y_o[...] = log_y
    u_s = eps_s_ref[...] * jnp.exp(0.5 * log_s) + mu_s
    u_y = eps_y_ref[...] * jnp.exp(0.5 * log_y) + mu_y
    lat_o[...] = jnp.concatenate([u_s, u_y], axis=1)
    s1 = jax.nn.relu(mm(A, mm(u_s, wp256[_SD1:_SD1 + LS, :]))
                     + bias("sd1", H))
    s2 = jax.nn.relu(mm(A, mm(s1, wpxs[:, 128:129])) + bias("sd2", 1))
    s_hat_o[...] = jax.nn.sigmoid(s2)
    xw3 = mm(u_s, wp256[_XD1A:_XD1A + LS, :]) \
        + mm(u_y, wp256[_XD1B:_XD1B + LY, :])
    xp1 = mm(A, xw3) + bias("xd1", H)
    xp = mm(A, mm(xp1, wpxs[:, 0:128])) + bias("xd2", F_IN)
    xp_o[...] = xp
    xw4 = mm(xp, wp512[_YD1A:_YD1A + F_IN, :]) \
        + mm(u_y, wp512[_YD1B:_YD1B + LY, :])
    yl1 = mm(A, xw4) + bias("yd1", 512)
    ylog = mm(A, mm(yl1, wpyy[:, 0:NL])) + bias("yd2", NL)
    ypred_o[...] = jax.nn.softmax(ylog, axis=1)
    yq1 = mm(A, mm(x, wp512[_YP1:_YP1 + F_IN, :])) + bias("yp1", 512)
    qlog = mm(A, mm(yq1, wpyy[:, NL:2 * NL])) + bias("yp2", NL)
    yprime_o[...] = jax.nn.softmax(qlog, axis=1)


def _run_fwd(edge_index, x, Y, eps_s, eps_y, params):
    f32 = jnp.float32
    outs = (
        jax.ShapeDtypeStruct((N, LS), f32),
        jax.ShapeDtypeStruct((N, LS), f32),
        jax.ShapeDtypeStruct((N, LY), f32),
        jax.ShapeDtypeStruct((N, LY), f32),
        jax.ShapeDtypeStruct((N, 1), f32),
        jax.ShapeDtypeStruct((N, F_IN), f32),
        jax.ShapeDtypeStruct((N, NL), f32),
        jax.ShapeDtypeStruct((N, NL), f32),
        jax.ShapeDtypeStruct((N, LS + LY), f32),
    )
    return pl.pallas_call(
        _fwd_body,
        out_shape=outs,
        interpret=_INTERPRET,
    )(edge_index, x, Y, eps_s, eps_y, *params)


def _fc_body(sched_ref, f_ref, w1_ref, b1_ref, w2_ref, b2_ref,
             l_ref, a_ref, l1_ref):
    j = pl.program_id(0)

    @pl.when(j == 0)
    def _():
        l1_ref[...] = b1_ref[...]

    @pl.when(j < P1)
    def _():
        l1_ref[...] += jnp.dot(f_ref[...], w1_ref[...],
                               preferred_element_type=jnp.float32)

    @pl.when(j >= P1)
    def _():
        b = j - P1
        val = jnp.dot(l1_ref[...], w2_ref[...],
                      preferred_element_type=jnp.float32) + b2_ref[...]
        l_ref[0:1, pl.ds(pl.multiple_of(b * CB2, 128), CB2)] = val
        iota_l = jax.lax.broadcasted_iota(jnp.int32, (1, N), 1)

        def rowbody(i, _):
            start = i * (i + 1) // 2
            base = jnp.minimum((start // 128) * 128, NUM_EDGES - WIN)
            base = pl.multiple_of(base, 128)
            off = start - base
            w = l_ref[0:1, pl.ds(base, WIN)]
            rowv = pltpu.roll(w, (WIN - off) % WIN, axis=1)[:, :N]
            a_ref[pl.ds(i, 1), :] = jnp.where(iota_l <= i, rowv, 0.0)
            return 0

        jax.lax.fori_loop(sched_ref[j], sched_ref[j + 1], rowbody, 0)


def _run_fc(sched, feat, W1, b1, W2, b2):
    f32 = jnp.float32
    return pl.pallas_call(
        _fc_body,
        grid=(P1 + P2,),
        in_specs=[
            pl.BlockSpec(memory_space=pltpu.SMEM),
            pl.BlockSpec((1, KB1), lambda j: (0, jnp.minimum(j, P1 - 1))),
            pl.BlockSpec((KB1, 512), lambda j: (jnp.minimum(j, P1 - 1), 0)),
            pl.BlockSpec((1, 512), lambda j: (0, 0)),
            pl.BlockSpec((512, CB2),
                         lambda j: (0, jnp.clip(j - P1, 0, P2 - 1))),
            pl.BlockSpec((1, CB2),
                         lambda j: (0, jnp.clip(j - P1, 0, P2 - 1))),
        ],
        out_specs=(
            pl.BlockSpec((1, NUM_EDGES), lambda j: (0, 0)),
            pl.BlockSpec((N, N), lambda j: (0, 0)),
        ),
        out_shape=(
            jax.ShapeDtypeStruct((1, NUM_EDGES), f32),
            jax.ShapeDtypeStruct((N, N), f32),
        ),
        scratch_shapes=[pltpu.VMEM((1, 512), f32)],
        interpret=_INTERPRET,
    )(sched, feat, W1, b1, W2, b2)


def kernel(x, edge_index, Y,
           us1_W, us1_b, usmu_W, usmu_b, uslog_W, uslog_b,
           uy1_W, uy1_b, uymu_W, uymu_b, uylog_W, uylog_b,
           sd1_W, sd1_b, sd2_W, sd2_b,
           xd1_W, xd1_b, xd2_W, xd2_b,
           fc1A_W, fc1A_b, fc2A_W, fc2A_b,
           yd1_W, yd1_b, yd2_W, yd2_b,
           yp1_W, yp1_b, yp2_W, yp2_b):
    f32 = jnp.float32
    row = lambda b: b.reshape(1, -1)
    eps_s = jax.random.normal(jax.random.key(42), (N, LS), dtype=f32)
    eps_y = jax.random.normal(jax.random.key(43), (N, LY), dtype=f32)

    z7 = jnp.zeros((7, H), f32)
    wp256 = jnp.concatenate(
        [us1_W, uy1_W[:F_IN], uy1_W[F_IN:F_IN + 1], z7,
         sd1_W, xd1_W[:LS], xd1_W[LS:]], axis=0)
    wp32 = jnp.concatenate([usmu_W, uslog_W, uymu_W, uylog_W],
                           axis=0)
    wp512 = jnp.concatenate([yd1_W[:F_IN], yd1_W[F_IN:], yp1_W],
                            axis=0)
    wpxs = jnp.concatenate([xd2_W, sd2_W], axis=1)
    wpyy = jnp.concatenate([yd2_W, yp2_W], axis=1)
    zf = lambda n: jnp.zeros((n,), f32)
    bp = jnp.concatenate(
        [us1_b, uy1_b, usmu_b, zf(96), uslog_b, zf(96), uymu_b, zf(96),
         uylog_b, zf(96), sd1_b, sd2_b, zf(127), xd1_b, xd2_b,
         yd1_b, yd2_b, zf(120), yp1_b, yp2_b, zf(120)]).reshape(1, 3072)
    params = (wp256, wp32, wp512, wpxs, wpyy, bp)
    (mu_s, log_s, mu_y, log_y, s_hat, xp, ypred, yprime, lat) = _run_fwd(
        edge_index, x, Y, eps_s, eps_y, params)

    feat = lat.reshape(1, N * (LS + LY))
    sched = jnp.asarray(_tri_schedule())
    l2d, A = _run_fc(sched, feat, fc1A_W, row(fc1A_b), fc2A_W, row(fc2A_b))
    return (xp, A, l2d.reshape(NUM_EDGES), ypred, yprime,
            s_hat, mu_s, log_s, mu_y, log_y)
